# baseline (speedup 1.0000x reference)
import functools

import jax
import jax.numpy as jnp
from jax import lax
from jax.experimental import pallas as pl
from jax.experimental.pallas import tpu as pltpu
from jax.experimental.pallas import tpu_sc as plsc

N = 10000
D_IN = 256
D_OUT = 256
E = 160000
H = 128
NS = 16
K = 128
CH = 80
PC = 40
NP = CH // PC
EPT = CH * K
EPAD = EPT * NS
IDX_ROWS = NS * CH + 8
AGG_ROWS = ((N + NS - 1) // NS + K - 1) // K * K * NS
ZR = AGG_ROWS // NS

def _sc_body(y_hbm, row_hbm, col_hbm, out_hbm,
             row_all, col_all, buf0, buf1, agg_sh, sem0, sem1):
    c = lax.axis_index("c")
    s = lax.axis_index("s")

    zero16 = jnp.zeros((16,), jnp.float32)

    def _zrow(r, carry):
        for j in range(buf0.shape[1] // 16):
            buf0[r, pl.ds(j * 16, 16)] = zero16
        return carry

    lax.fori_loop(0, K, _zrow, 0)
    for i in range(ZR // K):
        pltpu.sync_copy(buf0, agg_sh.at[pl.ds(s * ZR + i * K, K), :])
    plsc.subcore_barrier()

    for p in range(NP):
        base = s * CH + p * PC
        pltpu.sync_copy(row_hbm.at[c, pl.ds(base, PC + 8), :], row_all)
        pltpu.sync_copy(col_hbm.at[pl.ds(base, PC + 8), :], col_all)
        pltpu.make_async_copy(y_hbm.at[row_all.at[0]], buf0, sem0).start()
        pltpu.make_async_copy(y_hbm.at[row_all.at[1]], buf1, sem1).start()

        def _pair(i2, carry):
            i = i2 * 2
            pltpu.make_async_copy(y_hbm.at[row_all.at[i]], buf0, sem0).wait()
            pltpu.sync_copy(buf0, agg_sh.at[col_all.at[i]], add=True)
            pltpu.make_async_copy(y_hbm.at[row_all.at[i + 2]], buf0, sem0).start()
            pltpu.make_async_copy(y_hbm.at[row_all.at[i + 1]], buf1, sem1).wait()
            pltpu.sync_copy(buf1, agg_sh.at[col_all.at[i + 1]], add=True)
            pltpu.make_async_copy(y_hbm.at[row_all.at[i + 3]], buf1, sem1).start()
            return carry

        lax.fori_loop(0, PC // 2, _pair, 0)
        pltpu.make_async_copy(y_hbm.at[row_all.at[0]], buf0, sem0).wait()
        pltpu.make_async_copy(y_hbm.at[row_all.at[1]], buf1, sem1).wait()
    plsc.subcore_barrier()

    for i in range(ZR // K):
        r0 = s * ZR + i * K
        buf = buf0 if i % 2 == 0 else buf1
        pltpu.sync_copy(agg_sh.at[pl.ds(r0, K), :], buf)
        pltpu.sync_copy(buf, out_hbm.at[pl.ds(c * AGG_ROWS + r0, K), :])


def _sc_gather_scatter(y2, row_pad, col_pad):
    mesh = plsc.VectorSubcoreMesh(core_axis_name="c", subcore_axis_name="s")
    f = functools.partial(
        pl.kernel,
        mesh=mesh,
        out_type=jax.ShapeDtypeStruct((2 * AGG_ROWS, H), jnp.float32),
        scratch_types=[
            pltpu.VMEM((PC + 8, K), jnp.int32),
            pltpu.VMEM((PC + 8, K), jnp.int32),
            pltpu.VMEM((K, H), jnp.float32),
            pltpu.VMEM((K, H), jnp.float32),
            pltpu.VMEM_SHARED((AGG_ROWS, H), jnp.float32),
            pltpu.SemaphoreType.DMA,
            pltpu.SemaphoreType.DMA,
        ],
    )(_sc_body)
    return f(y2, row_pad, col_pad)


def _scale_body(x_ref, d_ref, o_ref):
    v = x_ref[...] * d_ref[...]
    o_ref[:, 0, :] = v[:, :H]
    o_ref[:, 1, :] = v[:, H:]


def _epilogue_body(a0_ref, a1_ref, w_ref, b_ref, d_ref, o_ref):
    acc = lax.dot_general(a0_ref[...], w_ref[:, :H], (((1,), (1,)), ((), ())),
                          preferred_element_type=jnp.float32)
    acc = acc + lax.dot_general(a1_ref[...], w_ref[:, H:], (((1,), (1,)), ((), ())),
                                preferred_element_type=jnp.float32)
    v = acc * d_ref[...] + b_ref[...]
    o_ref[...] = jnp.where(v > 0, v, jnp.exp(jnp.minimum(v, 0.0)) - 1.0)


def kernel(x, edge_index, deg_inv_sqrt, W, b):
    row = edge_index[0].astype(jnp.int32)
    col = edge_index[1].astype(jnp.int32)
    row_pad = jnp.concatenate(
        [row, jnp.zeros((EPAD - E,), jnp.int32)]).reshape(NS * CH, K)
    col_pad = jnp.concatenate(
        [col, jnp.full((EPAD - E,), N, jnp.int32)]).reshape(NS * CH, K)
    row_pad = jnp.concatenate([row_pad, jnp.zeros((8, K), jnp.int32)])
    col_pad = jnp.concatenate([col_pad, jnp.full((8, K), N, jnp.int32)])
    row2 = jnp.stack([2 * row_pad, 2 * row_pad + 1])
    d2 = deg_inv_sqrt.reshape(N, 1)

    NB = 2000
    nb = N // NB
    y3 = pl.pallas_call(
        _scale_body,
        grid=(nb,),
        in_specs=[
            pl.BlockSpec((NB, D_IN), lambda i: (i, 0)),
            pl.BlockSpec((NB, 1), lambda i: (i, 0)),
        ],
        out_specs=pl.BlockSpec((NB, 2, H), lambda i: (i, 0, 0)),
        out_shape=jax.ShapeDtypeStruct((N, 2, H), jnp.float32),
    )(x, d2)
    y2 = y3.reshape(2 * N, H)

    s2 = _sc_gather_scatter(y2, row2, col_pad)

    MB = 640
    mb = AGG_ROWS // MB
    d2p = jnp.pad(d2, ((0, AGG_ROWS - N), (0, 0)))
    out = pl.pallas_call(
        _epilogue_body,
        grid=(mb,),
        in_specs=[
            pl.BlockSpec((MB, H), lambda i: (i, 0)),
            pl.BlockSpec((MB, H), lambda i: (mb + i, 0)),
            pl.BlockSpec((D_OUT, D_IN), lambda i: (0, 0)),
            pl.BlockSpec((1, D_OUT), lambda i: (0, 0)),
            pl.BlockSpec((MB, 1), lambda i: (i, 0)),
        ],
        out_specs=pl.BlockSpec((MB, D_OUT), lambda i: (i, 0)),
        out_shape=jax.ShapeDtypeStruct((AGG_ROWS, D_OUT), jnp.float32),
    )(s2, s2, W, b.reshape(1, D_OUT), d2p)
    return out[:N]


# speedup vs baseline: 1.0877x; 1.0877x over previous
import functools

import jax
import jax.numpy as jnp
from jax import lax
from jax.experimental import pallas as pl
from jax.experimental.pallas import tpu as pltpu
from jax.experimental.pallas import tpu_sc as plsc

N = 10000
D_IN = 256
D_OUT = 256
E = 160000
H = 128
NS = 16
K = 128
CH = 80
PC = 40
NP = CH // PC
EPT = CH * K
EPAD = EPT * NS
IDX_ROWS = NS * CH + 8
AGG_ROWS = ((N + NS - 1) // NS + K - 1) // K * K * NS
ZR = AGG_ROWS // NS

def _sc_body(y_hbm, row_hbm, col_hbm, out_hbm,
             row_all, col_all, buf0, buf1, agg_sh, sem0, sem1):
    c = lax.axis_index("c")
    s = lax.axis_index("s")

    zero16 = jnp.zeros((16,), jnp.float32)

    def _zrow(r, carry):
        for j in range(buf0.shape[1] // 16):
            buf0[r, pl.ds(j * 16, 16)] = zero16
        return carry

    lax.fori_loop(0, K, _zrow, 0)
    for i in range(ZR // K):
        pltpu.sync_copy(buf0, agg_sh.at[pl.ds(s * ZR + i * K, K), :])
    plsc.subcore_barrier()

    for p in range(NP):
        base = s * CH + p * PC
        pltpu.sync_copy(row_hbm.at[c, pl.ds(base, PC + 8), :], row_all)
        pltpu.sync_copy(col_hbm.at[pl.ds(base, PC + 8), :], col_all)
        pltpu.make_async_copy(y_hbm.at[row_all.at[0]], buf0, sem0).start()
        pltpu.make_async_copy(y_hbm.at[row_all.at[1]], buf1, sem1).start()

        def _pair(i2, carry):
            i = i2 * 2
            pltpu.make_async_copy(y_hbm.at[row_all.at[i]], buf0, sem0).wait()
            pltpu.sync_copy(buf0, agg_sh.at[col_all.at[i]], add=True)
            pltpu.make_async_copy(y_hbm.at[row_all.at[i + 2]], buf0, sem0).start()
            pltpu.make_async_copy(y_hbm.at[row_all.at[i + 1]], buf1, sem1).wait()
            pltpu.sync_copy(buf1, agg_sh.at[col_all.at[i + 1]], add=True)
            pltpu.make_async_copy(y_hbm.at[row_all.at[i + 3]], buf1, sem1).start()
            return carry

        lax.fori_loop(0, PC // 2, _pair, 0)
        pltpu.make_async_copy(y_hbm.at[row_all.at[0]], buf0, sem0).wait()
        pltpu.make_async_copy(y_hbm.at[row_all.at[1]], buf1, sem1).wait()
    plsc.subcore_barrier()

    for i in range(ZR // K):
        r0 = s * ZR + i * K
        buf = buf0 if i % 2 == 0 else buf1
        pltpu.sync_copy(agg_sh.at[pl.ds(r0, K), :], buf)
        pltpu.sync_copy(buf, out_hbm.at[pl.ds(c * AGG_ROWS + r0, K), :])


def _sc_gather_scatter(y2, row_pad, col_pad):
    mesh = plsc.VectorSubcoreMesh(core_axis_name="c", subcore_axis_name="s")
    f = functools.partial(
        pl.kernel,
        mesh=mesh,
        out_type=jax.ShapeDtypeStruct((2 * AGG_ROWS, H), jnp.float32),
        scratch_types=[
            pltpu.VMEM((PC + 8, K), jnp.int32),
            pltpu.VMEM((PC + 8, K), jnp.int32),
            pltpu.VMEM((K, H), jnp.float32),
            pltpu.VMEM((K, H), jnp.float32),
            pltpu.VMEM_SHARED((AGG_ROWS, H), jnp.float32),
            pltpu.SemaphoreType.DMA,
            pltpu.SemaphoreType.DMA,
        ],
    )(_sc_body)
    return f(y2, row_pad, col_pad)


def _scale_body(x_ref, d_ref, o_ref):
    o_ref[...] = x_ref[...] * d_ref[...]


def _epilogue_body(a0_ref, a1_ref, w_ref, b_ref, d_ref, o_ref):
    acc = lax.dot_general(a0_ref[...], w_ref[:, :H], (((1,), (1,)), ((), ())),
                          preferred_element_type=jnp.float32)
    acc = acc + lax.dot_general(a1_ref[...], w_ref[:, H:], (((1,), (1,)), ((), ())),
                                preferred_element_type=jnp.float32)
    v = acc * d_ref[...] + b_ref[...]
    o_ref[...] = jnp.where(v > 0, v, jnp.exp(jnp.minimum(v, 0.0)) - 1.0)


def kernel(x, edge_index, deg_inv_sqrt, W, b):
    row = edge_index[0].astype(jnp.int32)
    col = edge_index[1].astype(jnp.int32)
    row_pad = jnp.concatenate(
        [row, jnp.zeros((EPAD - E,), jnp.int32)]).reshape(NS * CH, K)
    col_pad = jnp.concatenate(
        [col, jnp.full((EPAD - E,), N, jnp.int32)]).reshape(NS * CH, K)
    row_pad = jnp.concatenate([row_pad, jnp.zeros((8, K), jnp.int32)])
    col_pad = jnp.concatenate([col_pad, jnp.full((8, K), N, jnp.int32)])
    row2 = jnp.stack([row_pad, row_pad + N])
    d2 = deg_inv_sqrt.reshape(N, 1)

    NB = 2000
    nb = N // NB
    y2 = pl.pallas_call(
        _scale_body,
        grid=(2, nb),
        in_specs=[
            pl.BlockSpec((NB, H), lambda c, i: (i, c)),
            pl.BlockSpec((NB, 1), lambda c, i: (i, 0)),
        ],
        out_specs=pl.BlockSpec((NB, H), lambda c, i: (c * nb + i, 0)),
        out_shape=jax.ShapeDtypeStruct((2 * N, H), jnp.float32),
    )(x, d2)

    s2 = _sc_gather_scatter(y2, row2, col_pad)

    MB = 640
    mb = AGG_ROWS // MB
    d2p = jnp.pad(d2, ((0, AGG_ROWS - N), (0, 0)))
    out = pl.pallas_call(
        _epilogue_body,
        grid=(mb,),
        in_specs=[
            pl.BlockSpec((MB, H), lambda i: (i, 0)),
            pl.BlockSpec((MB, H), lambda i: (mb + i, 0)),
            pl.BlockSpec((D_OUT, D_IN), lambda i: (0, 0)),
            pl.BlockSpec((1, D_OUT), lambda i: (0, 0)),
            pl.BlockSpec((MB, 1), lambda i: (i, 0)),
        ],
        out_specs=pl.BlockSpec((MB, D_OUT), lambda i: (i, 0)),
        out_shape=jax.ShapeDtypeStruct((N, D_OUT), jnp.float32),
    )(s2, s2, W, b.reshape(1, D_OUT), d2p)
    return out
